# baseline (speedup 1.0000x reference)
import functools

import jax
import jax.numpy as jnp
from jax import lax
from jax.experimental import pallas as pl
from jax.experimental.pallas import tpu as pltpu
from jax.experimental.pallas import tpu_sc as plsc

MAXTIME = 5.0
NUM_STEPS = 100
DIM = 1024
N, B = 4096, 4
L = 16


def _sc_interp_body(idx_hbm, alpha_hbm, pe_hbm, out_hbm,
                    idx_v, alpha_v, rows_v, out_v, sem):
    cid = lax.axis_index("c")
    sid = lax.axis_index("s")

    @pl.when(jnp.logical_and(cid == 0, sid == 0))
    def _():
        pltpu.sync_copy(idx_hbm, idx_v)
        pltpu.sync_copy(alpha_hbm, alpha_v)
        pltpu.async_copy(pe_hbm.at[idx_v], rows_v, sem).wait()
        for b in range(B):
            ab = alpha_v[b, :]
            for c in range(DIM // L):
                sl = pl.ds(c * L, L)
                f = rows_v[b, sl]
                v = f + ab * (rows_v[b + B, sl] - f)
                out_v[b, sl] = v
                out_v[b + B, sl] = v
        pltpu.sync_copy(out_v, out_hbm)


def _sc_interp(idx, alpha_rep, pe):
    mesh = plsc.VectorSubcoreMesh(core_axis_name="c", subcore_axis_name="s", num_cores=1)
    k = functools.partial(
        pl.kernel,
        mesh=mesh,
        out_type=jax.ShapeDtypeStruct((2 * B, DIM), jnp.float32),
        scratch_types=[
            pltpu.VMEM((2 * B,), jnp.int32),
            pltpu.VMEM((B, L), jnp.float32),
            pltpu.VMEM((2 * B, DIM), jnp.float32),
            pltpu.VMEM((2 * B, DIM), jnp.float32),
            pltpu.SemaphoreType.DMA,
        ],
    )(_sc_interp_body)
    return k(idx, alpha_rep, pe)


def _tc_add_body(x_ref, p_ref, o_ref):
    o_ref[...] = x_ref[...] + p_ref[...][None]


NR = N * B // 8


def _tc_add(xs3, pe_big, blk):
    return pl.pallas_call(
        _tc_add_body,
        grid=(NR // blk,),
        in_specs=[
            pl.BlockSpec((blk, 8, DIM), lambda i: (i, 0, 0)),
            pl.BlockSpec(memory_space=pltpu.VMEM),
        ],
        out_specs=pl.BlockSpec((blk, 8, DIM), lambda i: (i, 0, 0)),
        out_shape=jax.ShapeDtypeStruct((NR, 8, DIM), jnp.float32),
        compiler_params=pltpu.CompilerParams(vmem_limit_bytes=120 * 1024 * 1024),
    )(xs3, pe_big)


def kernel(xs, times, pe):
    t = jnp.clip(times, 0.0, MAXTIME) * ((NUM_STEPS - 1) / MAXTIME)
    t_floor = jnp.floor(t)
    fi = t_floor.astype(jnp.int32)
    ci = jnp.ceil(t).astype(jnp.int32)
    alpha = t - t_floor
    idx = jnp.concatenate([fi, ci])
    alpha_rep = jnp.broadcast_to(alpha[:, None], (B, L))
    pe_big = _sc_interp(idx, alpha_rep, pe)
    xs3 = xs.reshape(NR, 8, DIM)
    out3 = _tc_add(xs3, pe_big, 256)
    return out3.reshape(N, B, DIM)


# speedup vs baseline: 3.2995x; 3.2995x over previous
import functools

import jax
import jax.numpy as jnp
from jax import lax
from jax.experimental import pallas as pl
from jax.experimental.pallas import tpu as pltpu
from jax.experimental.pallas import tpu_sc as plsc

MAXTIME = 5.0
NUM_STEPS = 100
DIM = 1024
N, B = 4096, 4
L = 16


def _sc_interp_body(idx_hbm, alpha_hbm, pe_hbm, out_hbm,
                    idx_v, alpha_v, rows_v, out_v, sem, sem2):
    cid = lax.axis_index("c")
    sid = lax.axis_index("s")

    @pl.when(jnp.logical_and(cid == 0, sid == 0))
    def _():
        c1 = pltpu.make_async_copy(idx_hbm, idx_v, sem)
        c2 = pltpu.make_async_copy(alpha_hbm, alpha_v, sem2)
        c1.start()
        c2.start()
        c1.wait()
        c2.wait()
        pltpu.async_copy(pe_hbm.at[idx_v], rows_v, sem).wait()
        for b in range(B):
            ab = alpha_v[b, :]

            @plsc.parallel_loop(0, DIM // L, unroll=8)
            def _(c):
                sl = pl.ds(c * L, L)
                f = rows_v[b, sl]
                out_v[b, sl] = f + ab * (rows_v[b + B, sl] - f)
        pltpu.sync_copy(out_v, out_hbm)


def _sc_interp(idx, alpha_rep, pe):
    mesh = plsc.VectorSubcoreMesh(core_axis_name="c", subcore_axis_name="s")
    k = functools.partial(
        pl.kernel,
        mesh=mesh,
        out_type=jax.ShapeDtypeStruct((B, DIM), jnp.float32),
        scratch_types=[
            pltpu.VMEM((L,), jnp.int32),
            pltpu.VMEM((B, L), jnp.float32),
            pltpu.VMEM((L, DIM), jnp.float32),
            pltpu.VMEM((B, DIM), jnp.float32),
            pltpu.SemaphoreType.DMA,
            pltpu.SemaphoreType.DMA,
        ],
    )(_sc_interp_body)
    return k(idx, alpha_rep, pe)


def _tc_add_body(x_ref, p_ref, o_ref):
    o_ref[...] = x_ref[...] + p_ref[...][None]


def _tc_add(xs, pe_interp, blk):
    return pl.pallas_call(
        _tc_add_body,
        grid=(N // blk,),
        in_specs=[
            pl.BlockSpec((blk, B, DIM), lambda i: (i, 0, 0)),
            pl.BlockSpec(memory_space=pltpu.VMEM),
        ],
        out_specs=pl.BlockSpec((blk, B, DIM), lambda i: (i, 0, 0)),
        out_shape=jax.ShapeDtypeStruct((N, B, DIM), jnp.float32),
    )(xs, pe_interp)


def kernel(xs, times, pe):
    t = jnp.clip(times, 0.0, MAXTIME) * ((NUM_STEPS - 1) / MAXTIME)
    t_floor = jnp.floor(t)
    fi = t_floor.astype(jnp.int32)
    ci = jnp.ceil(t).astype(jnp.int32)
    alpha = t - t_floor
    idx = jnp.concatenate([fi, ci, jnp.zeros((L - 2 * B,), jnp.int32)])
    alpha_rep = jnp.broadcast_to(alpha[:, None], (B, L))
    pe_interp = _sc_interp(idx, alpha_rep, pe)
    return _tc_add(xs, pe_interp, 512)
